# baseline (speedup 1.0000x reference)
import functools

import jax
import jax.numpy as jnp
from jax import lax
from jax.experimental import pallas as pl
from jax.experimental.pallas import tpu as pltpu
from jax.experimental.pallas import tpu_sc as plsc

B, S, D = 2, 2048, 1024
T = B * S
HID = 4096
E = 8
TM = 1024
HT = 512
J = HID // HT
NT = T // TM + E
PADT = NT * TM

NC, NS = 2, 16
NW = NC * NS
TPW = T // NW
CH = 64
NCH = TPW // CH

FP = jax.lax.Precision.HIGHEST


def _router_body(x_ref, wr_ref, sel_ref, w_ref, cnt_ref):
    k = pl.program_id(0)
    x = x_ref[...]
    wr = wr_ref[...]
    logits = lax.dot_general(x, wr, (((1,), (1,)), ((), ())),
                             preferred_element_type=jnp.float32)
    m0 = jnp.max(logits, axis=1, keepdims=True)
    io = lax.broadcasted_iota(jnp.int32, logits.shape, 1)
    sel = jnp.min(jnp.where(logits >= m0, io, E), axis=1, keepdims=True)
    v1 = jnp.max(jnp.where(io == sel, -jnp.inf, logits), axis=1, keepdims=True)
    w0 = 1.0 / (1.0 + jnp.exp(v1 - m0))
    sel_ref[...] = sel
    w_ref[...] = w0
    onehot = (io == sel).astype(jnp.float32)
    c = jnp.sum(onehot, axis=0, keepdims=True)

    @pl.when(k == 0)
    def _():
        cnt_ref[...] = c

    @pl.when(k > 0)
    def _():
        cnt_ref[...] += c


def _router(flat, wr):
    TK = 1024
    return pl.pallas_call(
        _router_body,
        grid=(T // TK,),
        in_specs=[
            pl.BlockSpec((TK, D), lambda k: (k, 0)),
            pl.BlockSpec((E, D), lambda k: (0, 0)),
        ],
        out_specs=[
            pl.BlockSpec((TK, 1), lambda k: (k, 0)),
            pl.BlockSpec((TK, 1), lambda k: (k, 0)),
            pl.BlockSpec((1, E), lambda k: (0, 0)),
        ],
        out_shape=[
            jax.ShapeDtypeStruct((T, 1), jnp.int32),
            jax.ShapeDtypeStruct((T, 1), jnp.float32),
            jax.ShapeDtypeStruct((1, E), jnp.float32),
        ],
    )(flat, wr)


def _dest_body(sel_ref, cnt_ref, dest_ref, crun_ref):
    k = pl.program_id(0)

    @pl.when(k == 0)
    def _():
        crun_ref[...] = jnp.zeros_like(crun_ref)

    ctot = cnt_ref[...]
    aligned = jnp.floor((ctot + (TM - 1)) * (1.0 / TM)) * TM
    r8 = lax.broadcasted_iota(jnp.int32, (E, E), 0)
    c8 = lax.broadcasted_iota(jnp.int32, (E, E), 1)
    upper = (r8 < c8).astype(jnp.float32)
    off = lax.dot_general(aligned, upper, (((1,), (0,)), ((), ())),
                          preferred_element_type=jnp.float32)
    base = off + crun_ref[...]

    sel = sel_ref[...]
    n = sel.shape[0]
    io = lax.broadcasted_iota(jnp.int32, (n, E), 1)
    onehot = (io == sel).astype(jnp.float32)
    rr = lax.broadcasted_iota(jnp.int32, (n, n), 0)
    cc = lax.broadcasted_iota(jnp.int32, (n, n), 1)
    lower = (cc < rr).astype(jnp.float32)
    rank = lax.dot_general(lower, onehot, (((1,), (0,)), ((), ())),
                           preferred_element_type=jnp.float32)
    destf = jnp.sum(onehot * (base + rank), axis=1, keepdims=True)
    dest_ref[...] = destf.astype(jnp.int32)
    crun_ref[...] = crun_ref[...] + jnp.sum(onehot, axis=0, keepdims=True)


def _dest_slots(sel, counts):
    TK2 = 512
    return pl.pallas_call(
        _dest_body,
        grid=(T // TK2,),
        in_specs=[
            pl.BlockSpec((TK2, 1), lambda k: (k, 0)),
            pl.BlockSpec((1, E), lambda k: (0, 0)),
        ],
        out_specs=pl.BlockSpec((TK2, 1), lambda k: (k, 0)),
        out_shape=jax.ShapeDtypeStruct((T, 1), jnp.int32),
        scratch_shapes=[pltpu.VMEM((1, E), jnp.float32)],
    )(sel, counts)


def _sc_scatter_body(flat_hbm, dest2_hbm, packed_hbm, idx_v, rows_v, sem):
    wid = lax.axis_index("s") * NC + lax.axis_index("c")
    for cc in range(NCH):
        r = wid * NCH + cc
        pltpu.sync_copy(dest2_hbm.at[r], idx_v)
        pltpu.sync_copy(flat_hbm.at[pl.ds(r * CH, CH)], rows_v)
        pltpu.async_copy(rows_v, packed_hbm.at[idx_v], sem).wait()


def _sc_scatter(flat, dest2):
    return pl.kernel(
        _sc_scatter_body,
        out_type=jax.ShapeDtypeStruct((PADT, D), jnp.float32),
        mesh=plsc.VectorSubcoreMesh(core_axis_name="c", subcore_axis_name="s"),
        scratch_types=[
            pltpu.VMEM((CH,), jnp.int32),
            pltpu.VMEM((CH, D), jnp.float32),
            pltpu.SemaphoreType.DMA,
        ],
    )(flat, dest2)


def _sc_gather_body(yp_hbm, dest2_hbm, out_hbm, idx_v, rows_v, sem):
    wid = lax.axis_index("s") * NC + lax.axis_index("c")
    for cc in range(NCH):
        r = wid * NCH + cc
        pltpu.sync_copy(dest2_hbm.at[r], idx_v)
        pltpu.async_copy(yp_hbm.at[idx_v], rows_v, sem).wait()
        pltpu.sync_copy(rows_v, out_hbm.at[pl.ds(r * CH, CH)])


def _sc_gather(yp, dest2):
    return pl.kernel(
        _sc_gather_body,
        out_type=jax.ShapeDtypeStruct((T, D), jnp.float32),
        mesh=plsc.VectorSubcoreMesh(core_axis_name="c", subcore_axis_name="s"),
        scratch_types=[
            pltpu.VMEM((CH,), jnp.int32),
            pltpu.VMEM((CH, D), jnp.float32),
            pltpu.SemaphoreType.DMA,
        ],
    )(yp, dest2)


def _gelu_exact(h):
    return h * 0.5 * (1.0 + lax.erf(h * (2.0 ** -0.5)))


def _ffn_body(eot_ref, nreal_ref, x_ref, w1_ref, w2_ref, y_ref):
    i = pl.program_id(0)
    j = pl.program_id(1)
    live = i < nreal_ref[0]

    @pl.when(live)
    def _():
        x = x_ref[...].astype(jnp.bfloat16)
        w1 = w1_ref[0].astype(jnp.bfloat16)
        h = lax.dot_general(x, w1, (((1,), (1,)), ((), ())),
                            preferred_element_type=jnp.float32)
        h = _gelu_exact(h).astype(jnp.bfloat16)
        w2 = w2_ref[0].astype(jnp.bfloat16)
        yj = lax.dot_general(h, w2, (((1,), (1,)), ((), ())),
                             preferred_element_type=jnp.float32)

        @pl.when(j == 0)
        def _():
            y_ref[...] = yj

        @pl.when(j > 0)
        def _():
            y_ref[...] += yj


def _grouped_ffn(eot, nreal, packed, w1, w2):
    def phys(i, nr):
        return jnp.minimum(i, nr[0] - 1)

    grid_spec = pltpu.PrefetchScalarGridSpec(
        num_scalar_prefetch=2,
        grid=(NT, J),
        in_specs=[
            pl.BlockSpec((TM, D), lambda i, j, eot, nr: (phys(i, nr), 0)),
            pl.BlockSpec(
                (1, HT, D),
                lambda i, j, eot, nr: (eot[phys(i, nr)],
                                       jnp.where(i < nr[0], j, J - 1), 0)),
            pl.BlockSpec(
                (1, D, HT),
                lambda i, j, eot, nr: (eot[phys(i, nr)], 0,
                                       jnp.where(i < nr[0], j, J - 1))),
        ],
        out_specs=pl.BlockSpec((TM, D), lambda i, j, eot, nr: (phys(i, nr), 0)),
    )
    return pl.pallas_call(
        _ffn_body,
        grid_spec=grid_spec,
        out_shape=jax.ShapeDtypeStruct((PADT, D), jnp.float32),
    )(eot, nreal, packed, w1, w2)


def _scale_body(y_ref, w_ref, o_ref):
    o_ref[...] = y_ref[...] * w_ref[...]


def _scale(yus, w0):
    TK = 1024
    return pl.pallas_call(
        _scale_body,
        grid=(T // TK,),
        in_specs=[
            pl.BlockSpec((TK, D), lambda k: (k, 0)),
            pl.BlockSpec((TK, 1), lambda k: (k, 0)),
        ],
        out_specs=pl.BlockSpec((TK, D), lambda k: (k, 0)),
        out_shape=jax.ShapeDtypeStruct((T, D), jnp.float32),
    )(yus, w0)


def kernel(x, Wr, W1, W2):
    flat = x.reshape(T, D)
    sel, w0, counts = _router(flat, Wr)

    counts_i = counts.reshape(E).astype(jnp.int32)
    aligned = ((counts_i + TM - 1) // TM) * TM
    cum = jnp.cumsum(aligned)
    nreal = (cum[-1] // TM).astype(jnp.int32).reshape(1)
    tile_start = jnp.arange(NT, dtype=jnp.int32) * TM
    eot = jnp.sum((tile_start[:, None] >= cum[None, :]).astype(jnp.int32),
                  axis=1)

    dest = _dest_slots(sel, counts)
    dest2 = dest.reshape(NW * NCH, CH)

    packed = _sc_scatter(flat, dest2)
    yp = _grouped_ffn(eot, nreal, packed, W1, W2)
    yus = _sc_gather(yp, dest2)
    out = _scale(yus, w0)
    return out.reshape(B, S, D)


# speedup vs baseline: 1.3608x; 1.3608x over previous
import functools

import jax
import jax.numpy as jnp
from jax import lax
from jax.experimental import pallas as pl
from jax.experimental.pallas import tpu as pltpu
from jax.experimental.pallas import tpu_sc as plsc

B, S, D = 2, 2048, 1024
T = B * S
HID = 4096
E = 8
TM = 768
HT = 1024
J = HID // HT
NT = -(-(T + E * (TM - 1)) // TM)

PADT = NT * TM

NC, NS = 2, 16
NW = NC * NS
TPW = T // NW
CH = 64
WREP = 128
NCH = TPW // CH


def _router_body(x_ref, wr_ref, sel_ref, w_ref, cnt_ref):
    k = pl.program_id(0)
    x = x_ref[...]
    wr = wr_ref[...]
    logits = lax.dot_general(x, wr, (((1,), (1,)), ((), ())),
                             preferred_element_type=jnp.float32)
    m0 = jnp.max(logits, axis=1, keepdims=True)
    io = lax.broadcasted_iota(jnp.int32, logits.shape, 1)
    sel = jnp.min(jnp.where(logits >= m0, io, E), axis=1, keepdims=True)
    v1 = jnp.max(jnp.where(io == sel, -jnp.inf, logits), axis=1, keepdims=True)
    w0 = 1.0 / (1.0 + jnp.exp(v1 - m0))
    sel_ref[...] = sel
    w_ref[...] = jnp.broadcast_to(w0, (w0.shape[0], WREP))
    onehot = (io == sel).astype(jnp.float32)
    c = jnp.sum(onehot, axis=0, keepdims=True)

    @pl.when(k == 0)
    def _():
        cnt_ref[...] = c

    @pl.when(k > 0)
    def _():
        cnt_ref[...] += c


def _router(flat, wr):
    TK = 1024
    return pl.pallas_call(
        _router_body,
        grid=(T // TK,),
        in_specs=[
            pl.BlockSpec((TK, D), lambda k: (k, 0)),
            pl.BlockSpec((E, D), lambda k: (0, 0)),
        ],
        out_specs=[
            pl.BlockSpec((TK, 1), lambda k: (k, 0)),
            pl.BlockSpec((TK, WREP), lambda k: (k, 0)),
            pl.BlockSpec((1, E), lambda k: (0, 0)),
        ],
        out_shape=[
            jax.ShapeDtypeStruct((T, 1), jnp.int32),
            jax.ShapeDtypeStruct((T, WREP), jnp.float32),
            jax.ShapeDtypeStruct((1, E), jnp.float32),
        ],
    )(flat, wr)


def _dest_body(sel_ref, cnt_ref, dest_ref, crun_ref):
    k = pl.program_id(0)

    @pl.when(k == 0)
    def _():
        crun_ref[...] = jnp.zeros_like(crun_ref)

    ctot = cnt_ref[...]
    aligned = jnp.floor((ctot + (TM - 1)) * (1.0 / TM)) * TM
    r8 = lax.broadcasted_iota(jnp.int32, (E, E), 0)
    c8 = lax.broadcasted_iota(jnp.int32, (E, E), 1)
    upper = (r8 < c8).astype(jnp.float32)
    off = lax.dot_general(aligned, upper, (((1,), (0,)), ((), ())),
                          preferred_element_type=jnp.float32)
    base = off + crun_ref[...]

    sel = sel_ref[...]
    n = sel.shape[0]
    io = lax.broadcasted_iota(jnp.int32, (n, E), 1)
    onehot = (io == sel).astype(jnp.float32)
    rr = lax.broadcasted_iota(jnp.int32, (n, n), 0)
    cc = lax.broadcasted_iota(jnp.int32, (n, n), 1)
    lower = (cc < rr).astype(jnp.float32)
    rank = lax.dot_general(lower, onehot, (((1,), (0,)), ((), ())),
                           preferred_element_type=jnp.float32)
    destf = jnp.sum(onehot * (base + rank), axis=1, keepdims=True)
    dest_ref[...] = destf.astype(jnp.int32)
    crun_ref[...] = crun_ref[...] + jnp.sum(onehot, axis=0, keepdims=True)


def _dest_slots(sel, counts):
    TK2 = 512
    return pl.pallas_call(
        _dest_body,
        grid=(T // TK2,),
        in_specs=[
            pl.BlockSpec((TK2, 1), lambda k: (k, 0)),
            pl.BlockSpec((1, E), lambda k: (0, 0)),
        ],
        out_specs=pl.BlockSpec((TK2, 1), lambda k: (k, 0)),
        out_shape=jax.ShapeDtypeStruct((T, 1), jnp.int32),
        scratch_shapes=[pltpu.VMEM((1, E), jnp.float32)],
    )(sel, counts)


def _sc_scatter_body(flat_hbm, dest2_hbm, w16_hbm, packed_hbm, packedw_hbm,
                     idx_v, rows_v, wrows_v, sem):
    wid = lax.axis_index("s") * NC + lax.axis_index("c")
    for cc in range(NCH):
        r = wid * NCH + cc
        pltpu.sync_copy(dest2_hbm.at[r], idx_v)
        pltpu.sync_copy(w16_hbm.at[pl.ds(r * CH, CH)], wrows_v)
        pltpu.sync_copy(flat_hbm.at[pl.ds(r * CH, CH)], rows_v)
        pltpu.async_copy(rows_v, packed_hbm.at[idx_v], sem).wait()
        pltpu.async_copy(wrows_v, packedw_hbm.at[idx_v], sem).wait()


def _sc_scatter(flat, dest2, w16):
    return pl.kernel(
        _sc_scatter_body,
        out_type=[
            jax.ShapeDtypeStruct((PADT, D), jnp.float32),
            jax.ShapeDtypeStruct((PADT, WREP), jnp.float32),
        ],
        mesh=plsc.VectorSubcoreMesh(core_axis_name="c", subcore_axis_name="s"),
        scratch_types=[
            pltpu.VMEM((CH,), jnp.int32),
            pltpu.VMEM((CH, D), jnp.float32),
            pltpu.VMEM((CH, WREP), jnp.float32),
            pltpu.SemaphoreType.DMA,
        ],
    )(flat, dest2, w16)


def _sc_gather_body(yp_hbm, dest2_hbm, out_hbm, idx_v, rows_v, sem):
    wid = lax.axis_index("s") * NC + lax.axis_index("c")
    for cc in range(NCH):
        r = wid * NCH + cc
        pltpu.sync_copy(dest2_hbm.at[r], idx_v)
        pltpu.async_copy(yp_hbm.at[idx_v], rows_v, sem).wait()
        pltpu.sync_copy(rows_v, out_hbm.at[pl.ds(r * CH, CH)])


def _sc_gather(yp, dest2):
    return pl.kernel(
        _sc_gather_body,
        out_type=jax.ShapeDtypeStruct((T, D), jnp.float32),
        mesh=plsc.VectorSubcoreMesh(core_axis_name="c", subcore_axis_name="s"),
        scratch_types=[
            pltpu.VMEM((CH,), jnp.int32),
            pltpu.VMEM((CH, D), jnp.float32),
            pltpu.SemaphoreType.DMA,
        ],
    )(yp, dest2)


def _gelu_exact(h):
    return h * 0.5 * (1.0 + lax.erf(h * (2.0 ** -0.5)))


def _ffn_body(eot_ref, nreal_ref, x_ref, w1_ref, w2_ref, wt_ref, y_ref):
    i = pl.program_id(0)
    j = pl.program_id(1)
    live = i < nreal_ref[0]

    @pl.when(live)
    def _():
        x = x_ref[...]
        w1 = w1_ref[0]
        h = lax.dot_general(x, w1, (((1,), (1,)), ((), ())),
                            preferred_element_type=jnp.float32)
        h = _gelu_exact(h)
        w2 = w2_ref[0]
        yj = lax.dot_general(h, w2, (((1,), (1,)), ((), ())),
                             preferred_element_type=jnp.float32)

        if J == 1:
            y_ref[...] = yj * wt_ref[:, 0:1]
        else:
            @pl.when(j == 0)
            def _():
                y_ref[...] = yj

            @pl.when(jnp.logical_and(j > 0, j < J - 1))
            def _():
                y_ref[...] += yj

            @pl.when(j == J - 1)
            def _():
                y_ref[...] = (y_ref[...] + yj) * wt_ref[:, 0:1]


def _grouped_ffn(eot, nreal, packed, packedw, w1, w2):
    def phys(i, nr):
        return jnp.minimum(i, nr[0] - 1)

    grid_spec = pltpu.PrefetchScalarGridSpec(
        num_scalar_prefetch=2,
        grid=(NT, J),
        in_specs=[
            pl.BlockSpec((TM, D), lambda i, j, eot, nr: (phys(i, nr), 0)),
            pl.BlockSpec(
                (1, HT, D),
                lambda i, j, eot, nr: (eot[phys(i, nr)],
                                       jnp.where(i < nr[0], j, J - 1), 0)),
            pl.BlockSpec(
                (1, D, HT),
                lambda i, j, eot, nr: (eot[phys(i, nr)], 0,
                                       jnp.where(i < nr[0], j, J - 1))),
            pl.BlockSpec((TM, WREP), lambda i, j, eot, nr: (phys(i, nr), 0)),
        ],
        out_specs=pl.BlockSpec((TM, D), lambda i, j, eot, nr: (phys(i, nr), 0)),
    )
    return pl.pallas_call(
        _ffn_body,
        grid_spec=grid_spec,
        out_shape=jax.ShapeDtypeStruct((PADT, D), jnp.float32),
    )(eot, nreal, packed, w1, w2, packedw)


def kernel(x, Wr, W1, W2):
    flat = x.reshape(T, D)
    sel, w16, counts = _router(flat, Wr)

    counts_i = counts.reshape(E).astype(jnp.int32)
    aligned = ((counts_i + TM - 1) // TM) * TM
    cum = jnp.cumsum(aligned)
    nreal = (cum[-1] // TM).astype(jnp.int32).reshape(1)
    tile_start = jnp.arange(NT, dtype=jnp.int32) * TM
    eot = jnp.sum((tile_start[:, None] >= cum[None, :]).astype(jnp.int32),
                  axis=1)

    dest = _dest_slots(sel, counts)
    dest2 = dest.reshape(NW * NCH, CH)

    packed, packedw = _sc_scatter(flat, dest2, w16)
    yp = _grouped_ffn(eot, nreal, packed, packedw, W1, W2)
    out = _sc_gather(yp, dest2)
    return out.reshape(B, S, D)


# speedup vs baseline: 1.6131x; 1.1854x over previous
import functools

import jax
import jax.numpy as jnp
from jax import lax
from jax.experimental import pallas as pl
from jax.experimental.pallas import tpu as pltpu
from jax.experimental.pallas import tpu_sc as plsc

B, S, D = 2, 2048, 1024
T = B * S
HID = 4096
E = 8
TM = 576
HT = 2048
J = HID // HT
NT = -(-(T + E * (TM - 1)) // TM)

PADT = NT * TM

NC, NS = 2, 16
NW = NC * NS
TPW = T // NW
CH = 64
WREP = 128
NCH = TPW // CH


def _router_body(x_ref, wr_ref, sel_ref, w_ref, cnt_ref):
    k = pl.program_id(0)
    x = x_ref[...]
    wr = wr_ref[...]
    logits = lax.dot_general(x, wr, (((1,), (1,)), ((), ())),
                             preferred_element_type=jnp.float32)
    m0 = jnp.max(logits, axis=1, keepdims=True)
    io = lax.broadcasted_iota(jnp.int32, logits.shape, 1)
    sel = jnp.min(jnp.where(logits >= m0, io, E), axis=1, keepdims=True)
    v1 = jnp.max(jnp.where(io == sel, -jnp.inf, logits), axis=1, keepdims=True)
    w0 = 1.0 / (1.0 + jnp.exp(v1 - m0))
    sel_ref[...] = sel
    w_ref[...] = jnp.broadcast_to(w0, (w0.shape[0], WREP))
    onehot = (io == sel).astype(jnp.float32)
    c = jnp.sum(onehot, axis=0, keepdims=True)

    @pl.when(k == 0)
    def _():
        cnt_ref[...] = c

    @pl.when(k > 0)
    def _():
        cnt_ref[...] += c


def _router(flat, wr):
    TK = 1024
    return pl.pallas_call(
        _router_body,
        grid=(T // TK,),
        in_specs=[
            pl.BlockSpec((TK, D), lambda k: (k, 0)),
            pl.BlockSpec((E, D), lambda k: (0, 0)),
        ],
        out_specs=[
            pl.BlockSpec((TK, 1), lambda k: (k, 0)),
            pl.BlockSpec((TK, WREP), lambda k: (k, 0)),
            pl.BlockSpec((1, E), lambda k: (0, 0)),
        ],
        out_shape=[
            jax.ShapeDtypeStruct((T, 1), jnp.int32),
            jax.ShapeDtypeStruct((T, WREP), jnp.float32),
            jax.ShapeDtypeStruct((1, E), jnp.float32),
        ],
    )(flat, wr)


def _dest_body(sel_ref, cnt_ref, dest_ref, crun_ref):
    k = pl.program_id(0)

    @pl.when(k == 0)
    def _():
        crun_ref[...] = jnp.zeros_like(crun_ref)

    ctot = cnt_ref[...]
    aligned = jnp.floor((ctot + (TM - 1)) * (1.0 / TM)) * TM
    r8 = lax.broadcasted_iota(jnp.int32, (E, E), 0)
    c8 = lax.broadcasted_iota(jnp.int32, (E, E), 1)
    upper = (r8 < c8).astype(jnp.float32)
    off = lax.dot_general(aligned, upper, (((1,), (0,)), ((), ())),
                          preferred_element_type=jnp.float32)
    base = off + crun_ref[...]

    sel = sel_ref[...]
    n = sel.shape[0]
    io = lax.broadcasted_iota(jnp.int32, (n, E), 1)
    onehot = (io == sel).astype(jnp.float32)
    rr = lax.broadcasted_iota(jnp.int32, (n, n), 0)
    cc = lax.broadcasted_iota(jnp.int32, (n, n), 1)
    lower = (cc < rr).astype(jnp.float32)
    rank = lax.dot_general(lower, onehot, (((1,), (0,)), ((), ())),
                           preferred_element_type=jnp.float32)
    destf = jnp.sum(onehot * (base + rank), axis=1, keepdims=True)
    dest_ref[...] = destf.astype(jnp.int32)
    crun_ref[...] = crun_ref[...] + jnp.sum(onehot, axis=0, keepdims=True)


def _dest_slots(sel, counts):
    TK2 = 512
    return pl.pallas_call(
        _dest_body,
        grid=(T // TK2,),
        in_specs=[
            pl.BlockSpec((TK2, 1), lambda k: (k, 0)),
            pl.BlockSpec((1, E), lambda k: (0, 0)),
        ],
        out_specs=pl.BlockSpec((TK2, 1), lambda k: (k, 0)),
        out_shape=jax.ShapeDtypeStruct((T, 1), jnp.int32),
        scratch_shapes=[pltpu.VMEM((1, E), jnp.float32)],
    )(sel, counts)


def _sc_scatter_body(flat_hbm, dest2_hbm, w16_hbm, packed_hbm, packedw_hbm,
                     idx_v, rows_v, wrows_v, sem):
    wid = lax.axis_index("s") * NC + lax.axis_index("c")
    for cc in range(NCH):
        r = wid * NCH + cc
        pltpu.sync_copy(dest2_hbm.at[r], idx_v)
        pltpu.sync_copy(w16_hbm.at[pl.ds(r * CH, CH)], wrows_v)
        pltpu.sync_copy(flat_hbm.at[pl.ds(r * CH, CH)], rows_v)
        pltpu.async_copy(rows_v, packed_hbm.at[idx_v], sem).wait()
        pltpu.async_copy(wrows_v, packedw_hbm.at[idx_v], sem).wait()


def _sc_scatter(flat, dest2, w16):
    return pl.kernel(
        _sc_scatter_body,
        out_type=[
            jax.ShapeDtypeStruct((PADT, D), jnp.float32),
            jax.ShapeDtypeStruct((PADT, WREP), jnp.float32),
        ],
        mesh=plsc.VectorSubcoreMesh(core_axis_name="c", subcore_axis_name="s"),
        scratch_types=[
            pltpu.VMEM((CH,), jnp.int32),
            pltpu.VMEM((CH, D), jnp.float32),
            pltpu.VMEM((CH, WREP), jnp.float32),
            pltpu.SemaphoreType.DMA,
        ],
    )(flat, dest2, w16)


def _sc_gather_body(yp_hbm, dest2_hbm, out_hbm, idx_v, rows_v, sem):
    wid = lax.axis_index("s") * NC + lax.axis_index("c")
    for cc in range(NCH):
        r = wid * NCH + cc
        pltpu.sync_copy(dest2_hbm.at[r], idx_v)
        pltpu.async_copy(yp_hbm.at[idx_v], rows_v, sem).wait()
        pltpu.sync_copy(rows_v, out_hbm.at[pl.ds(r * CH, CH)])


def _sc_gather(yp, dest2):
    return pl.kernel(
        _sc_gather_body,
        out_type=jax.ShapeDtypeStruct((T, D), jnp.float32),
        mesh=plsc.VectorSubcoreMesh(core_axis_name="c", subcore_axis_name="s"),
        scratch_types=[
            pltpu.VMEM((CH,), jnp.int32),
            pltpu.VMEM((CH, D), jnp.float32),
            pltpu.SemaphoreType.DMA,
        ],
    )(yp, dest2)


def _gelu_exact(h):
    return h * 0.5 * (1.0 + lax.erf(h * (2.0 ** -0.5)))


def _ffn_body(eot_ref, nreal_ref, x_ref, w1_ref, w2_ref, wt_ref, y_ref):
    i = pl.program_id(0)
    j = pl.program_id(1)
    live = i < nreal_ref[0]

    @pl.when(live)
    def _():
        x = x_ref[...]
        w1 = w1_ref[0]
        h = lax.dot_general(x, w1, (((1,), (1,)), ((), ())),
                            preferred_element_type=jnp.float32)
        h = _gelu_exact(h)
        w2 = w2_ref[0]
        yj = lax.dot_general(h, w2, (((1,), (1,)), ((), ())),
                             preferred_element_type=jnp.float32)

        if J == 1:
            y_ref[...] = yj * wt_ref[:, 0:1]
        else:
            @pl.when(j == 0)
            def _():
                y_ref[...] = yj

            @pl.when(jnp.logical_and(j > 0, j < J - 1))
            def _():
                y_ref[...] += yj

            @pl.when(j == J - 1)
            def _():
                y_ref[...] = (y_ref[...] + yj) * wt_ref[:, 0:1]


def _grouped_ffn(eot, nreal, packed, packedw, w1, w2):
    def phys(i, nr):
        return jnp.minimum(i, nr[0] - 1)

    grid_spec = pltpu.PrefetchScalarGridSpec(
        num_scalar_prefetch=2,
        grid=(NT, J),
        in_specs=[
            pl.BlockSpec((TM, D), lambda i, j, eot, nr: (phys(i, nr), 0)),
            pl.BlockSpec(
                (1, HT, D),
                lambda i, j, eot, nr: (eot[phys(i, nr)],
                                       jnp.where(i < nr[0], j, J - 1), 0)),
            pl.BlockSpec(
                (1, D, HT),
                lambda i, j, eot, nr: (eot[phys(i, nr)], 0,
                                       jnp.where(i < nr[0], j, J - 1))),
            pl.BlockSpec((TM, WREP), lambda i, j, eot, nr: (phys(i, nr), 0)),
        ],
        out_specs=pl.BlockSpec((TM, D), lambda i, j, eot, nr: (phys(i, nr), 0)),
    )
    return pl.pallas_call(
        _ffn_body,
        grid_spec=grid_spec,
        out_shape=jax.ShapeDtypeStruct((PADT, D), jnp.float32),
    )(eot, nreal, packed, w1, w2, packedw)


def kernel(x, Wr, W1, W2):
    flat = x.reshape(T, D)
    sel, w16, counts = _router(flat, Wr)

    counts_i = counts.reshape(E).astype(jnp.int32)
    aligned = ((counts_i + TM - 1) // TM) * TM
    cum = jnp.cumsum(aligned)
    nreal = (cum[-1] // TM).astype(jnp.int32).reshape(1)
    tile_start = jnp.arange(NT, dtype=jnp.int32) * TM
    eot = jnp.sum((tile_start[:, None] >= cum[None, :]).astype(jnp.int32),
                  axis=1)

    dest = _dest_slots(sel, counts)
    dest2 = dest.reshape(NW * NCH, CH)

    packed, packedw = _sc_scatter(flat, dest2, w16)
    yp = _grouped_ffn(eot, nreal, packed, packedw, W1, W2)
    out = _sc_gather(yp, dest2)
    return out.reshape(B, S, D)


# speedup vs baseline: 1.6134x; 1.0002x over previous
import functools

import jax
import jax.numpy as jnp
from jax import lax
from jax.experimental import pallas as pl
from jax.experimental.pallas import tpu as pltpu
from jax.experimental.pallas import tpu_sc as plsc

B, S, D = 2, 2048, 1024
T = B * S
HID = 4096
E = 8
TM = 576
HT = 2048
J = HID // HT
NT = -(-(T + E * (TM - 1)) // TM)

PADT = NT * TM

NC, NS = 2, 16
NW = NC * NS
TPW = T // NW
CH = 64
WREP = 128
NCH = TPW // CH


TK = 512
NP = T // TK


def _router_body(x_ref, wr_ref, w_ref, dest_ref, eot_ref, nr_ref,
                 sel_s, cnt_s, crun_s):
    k = pl.program_id(0)

    @pl.when(k < NP)
    def _():
        x = x_ref[...]
        wr = wr_ref[...]
        logits = lax.dot_general(x, wr, (((1,), (1,)), ((), ())),
                                 preferred_element_type=jnp.float32)
        m0 = jnp.max(logits, axis=1, keepdims=True)
        io = lax.broadcasted_iota(jnp.int32, logits.shape, 1)
        sel = jnp.min(jnp.where(logits >= m0, io, E), axis=1, keepdims=True)
        v1 = jnp.max(jnp.where(io == sel, -jnp.inf, logits),
                     axis=1, keepdims=True)
        w0 = 1.0 / (1.0 + jnp.exp(v1 - m0))
        w_ref[...] = jnp.broadcast_to(w0, (TK, WREP))
        sel_s[pl.ds(k * TK, TK), :] = sel
        onehot = (io == sel).astype(jnp.float32)
        c = jnp.sum(onehot, axis=0, keepdims=True)

        @pl.when(k == 0)
        def _():
            cnt_s[...] = c

        @pl.when(k > 0)
        def _():
            cnt_s[...] += c

    @pl.when(k >= NP)
    def _():
        cb = k - NP

        @pl.when(k == NP)
        def _():
            crun_s[...] = jnp.zeros_like(crun_s)

        ctot = cnt_s[...]
        aligned = jnp.floor((ctot + (TM - 1)) * (1.0 / TM)) * TM
        r8 = lax.broadcasted_iota(jnp.int32, (E, E), 0)
        c8 = lax.broadcasted_iota(jnp.int32, (E, E), 1)
        upper = (r8 < c8).astype(jnp.float32)
        off = lax.dot_general(aligned, upper, (((1,), (0,)), ((), ())),
                              preferred_element_type=jnp.float32)
        base = off + crun_s[...]

        sel = sel_s[pl.ds(cb * TK, TK), :]
        io = lax.broadcasted_iota(jnp.int32, (TK, E), 1)
        onehot = (io == sel).astype(jnp.float32)
        rr = lax.broadcasted_iota(jnp.int32, (TK, TK), 0)
        cc = lax.broadcasted_iota(jnp.int32, (TK, TK), 1)
        lower = (cc < rr).astype(jnp.float32)
        rank = lax.dot_general(lower, onehot, (((1,), (0,)), ((), ())),
                               preferred_element_type=jnp.float32)
        destf = jnp.sum(onehot * (base + rank), axis=1, keepdims=True)
        dest_ref[...] = destf.astype(jnp.int32)
        crun_s[...] = crun_s[...] + jnp.sum(onehot, axis=0, keepdims=True)

        @pl.when(k == 2 * NP - 1)
        def _():
            ts = (lax.broadcasted_iota(jnp.int32, (128, E), 0)
                  .astype(jnp.float32) * TM)
            inreg = jnp.logical_and(ts >= off, ts < off + aligned)
            ef = lax.broadcasted_iota(jnp.int32, (128, E), 1).astype(jnp.float32)
            eot_ref[...] = jnp.sum(
                jnp.where(inreg, ef, 0.0), axis=1, keepdims=True
            ).astype(jnp.int32)
            total = jnp.sum(aligned, axis=1, keepdims=True)
            nr_ref[...] = (total * (1.0 / TM)).astype(jnp.int32)


def _router_dest(flat, wr):
    return pl.pallas_call(
        _router_body,
        grid=(2 * NP,),
        in_specs=[
            pl.BlockSpec((TK, D), lambda k: (jnp.minimum(k, NP - 1), 0)),
            pl.BlockSpec((E, D), lambda k: (0, 0)),
        ],
        out_specs=[
            pl.BlockSpec((TK, WREP), lambda k: (jnp.minimum(k, NP - 1), 0)),
            pl.BlockSpec((TK, 1), lambda k: (jnp.maximum(k - NP, 0), 0)),
            pl.BlockSpec((128, 1), lambda k: (0, 0)),
            pl.BlockSpec((1, 1), lambda k: (0, 0)),
        ],
        out_shape=[
            jax.ShapeDtypeStruct((T, WREP), jnp.float32),
            jax.ShapeDtypeStruct((T, 1), jnp.int32),
            jax.ShapeDtypeStruct((128, 1), jnp.int32),
            jax.ShapeDtypeStruct((1, 1), jnp.int32),
        ],
        scratch_shapes=[
            pltpu.VMEM((T, 1), jnp.int32),
            pltpu.VMEM((1, E), jnp.float32),
            pltpu.VMEM((1, E), jnp.float32),
        ],
    )(flat, wr)


def _sc_scatter_body(flat_hbm, dest2_hbm, w16_hbm, packed_hbm, packedw_hbm,
                     idx_v, rows_v, wrows_v, sem):
    wid = lax.axis_index("s") * NC + lax.axis_index("c")
    for cc in range(NCH):
        r = wid * NCH + cc
        pltpu.sync_copy(dest2_hbm.at[r], idx_v)
        pltpu.sync_copy(w16_hbm.at[pl.ds(r * CH, CH)], wrows_v)
        pltpu.sync_copy(flat_hbm.at[pl.ds(r * CH, CH)], rows_v)
        pltpu.async_copy(rows_v, packed_hbm.at[idx_v], sem).wait()
        pltpu.async_copy(wrows_v, packedw_hbm.at[idx_v], sem).wait()


def _sc_scatter(flat, dest2, w16):
    return pl.kernel(
        _sc_scatter_body,
        out_type=[
            jax.ShapeDtypeStruct((PADT, D), jnp.float32),
            jax.ShapeDtypeStruct((PADT, WREP), jnp.float32),
        ],
        mesh=plsc.VectorSubcoreMesh(core_axis_name="c", subcore_axis_name="s"),
        scratch_types=[
            pltpu.VMEM((CH,), jnp.int32),
            pltpu.VMEM((CH, D), jnp.float32),
            pltpu.VMEM((CH, WREP), jnp.float32),
            pltpu.SemaphoreType.DMA,
        ],
    )(flat, dest2, w16)


def _sc_gather_body(yp_hbm, dest2_hbm, out_hbm, idx_v, rows_v, sem):
    wid = lax.axis_index("s") * NC + lax.axis_index("c")
    for cc in range(NCH):
        r = wid * NCH + cc
        pltpu.sync_copy(dest2_hbm.at[r], idx_v)
        pltpu.async_copy(yp_hbm.at[idx_v], rows_v, sem).wait()
        pltpu.sync_copy(rows_v, out_hbm.at[pl.ds(r * CH, CH)])


def _sc_gather(yp, dest2):
    return pl.kernel(
        _sc_gather_body,
        out_type=jax.ShapeDtypeStruct((T, D), jnp.float32),
        mesh=plsc.VectorSubcoreMesh(core_axis_name="c", subcore_axis_name="s"),
        scratch_types=[
            pltpu.VMEM((CH,), jnp.int32),
            pltpu.VMEM((CH, D), jnp.float32),
            pltpu.SemaphoreType.DMA,
        ],
    )(yp, dest2)


def _gelu_exact(h):
    return h * 0.5 * (1.0 + lax.erf(h * (2.0 ** -0.5)))


def _ffn_body(eot_ref, nreal_ref, x_ref, w1_ref, w2_ref, wt_ref, y_ref):
    i = pl.program_id(0)
    j = pl.program_id(1)
    live = i < nreal_ref[0]

    @pl.when(live)
    def _():
        x = x_ref[...]
        w1 = w1_ref[0]
        h = lax.dot_general(x, w1, (((1,), (1,)), ((), ())),
                            preferred_element_type=jnp.float32)
        h = _gelu_exact(h)
        w2 = w2_ref[0]
        yj = lax.dot_general(h, w2, (((1,), (1,)), ((), ())),
                             preferred_element_type=jnp.float32)

        if J == 1:
            y_ref[...] = yj * wt_ref[:, 0:1]
        else:
            @pl.when(j == 0)
            def _():
                y_ref[...] = yj

            @pl.when(jnp.logical_and(j > 0, j < J - 1))
            def _():
                y_ref[...] += yj

            @pl.when(j == J - 1)
            def _():
                y_ref[...] = (y_ref[...] + yj) * wt_ref[:, 0:1]


def _grouped_ffn(eot, nreal, packed, packedw, w1, w2):
    def phys(i, nr):
        return jnp.minimum(i, nr[0] - 1)

    grid_spec = pltpu.PrefetchScalarGridSpec(
        num_scalar_prefetch=2,
        grid=(NT, J),
        in_specs=[
            pl.BlockSpec((TM, D), lambda i, j, eot, nr: (phys(i, nr), 0)),
            pl.BlockSpec(
                (1, HT, D),
                lambda i, j, eot, nr: (eot[phys(i, nr)],
                                       jnp.where(i < nr[0], j, J - 1), 0)),
            pl.BlockSpec(
                (1, D, HT),
                lambda i, j, eot, nr: (eot[phys(i, nr)], 0,
                                       jnp.where(i < nr[0], j, J - 1))),
            pl.BlockSpec((TM, WREP), lambda i, j, eot, nr: (phys(i, nr), 0)),
        ],
        out_specs=pl.BlockSpec((TM, D), lambda i, j, eot, nr: (phys(i, nr), 0)),
    )
    return pl.pallas_call(
        _ffn_body,
        grid_spec=grid_spec,
        out_shape=jax.ShapeDtypeStruct((PADT, D), jnp.float32),
    )(eot, nreal, packed, w1, w2, packedw)


def kernel(x, Wr, W1, W2):
    flat = x.reshape(T, D)
    w16, dest, eot128, nr11 = _router_dest(flat, Wr)
    eot = eot128[:NT, 0]
    nreal = nr11[0]
    dest2 = dest.reshape(NW * NCH, CH)

    packed, packedw = _sc_scatter(flat, dest2, w16)
    yp = _grouped_ffn(eot, nreal, packed, packedw, W1, W2)
    out = _sc_gather(yp, dest2)
    return out.reshape(B, S, D)
